# baseline (speedup 1.0000x reference)
import functools

import jax
import jax.numpy as jnp
from jax import lax
from jax.experimental import pallas as pl
from jax.experimental.pallas import tpu as pltpu
from jax.experimental.pallas import tpu_sc as plsc

C, P, G, S = 8, 16, 2000, 10000
G_SEL, S_SEL = 512, 2048

NC, NS = 2, 16
NW = NC * NS
L = 16

ROWS_PER_W = C * G_SEL // NW
GENE_BLOCKS = G_SEL // ROWS_PER_W
RB = 8
NBATCH = ROWS_PER_W // RB
W_ROWS_PER_W = C * P // NW
CHUNKS = S_SEL // L
UNROLL = 8


def _gather_row(spots_v, src_ref, dst_ref, r):
    rsplat = jnp.full((L,), r, jnp.int32)

    def body(j, _):
        base = j * (L * UNROLL)
        for u in range(UNROLL):
            off = base + u * L
            idx = spots_v[pl.ds(off, L)]
            dst_ref[r, pl.ds(off, L)] = plsc.load_gather(src_ref, [rsplat, idx])
        return 0
    lax.fori_loop(0, CHUNKS // UNROLL, body, 0)


def _sc_body(lr_hbm, w_hbm, v_hbm, spots_hbm, genes_hbm,
             lr_out, w_out, v_out,
             spots_v, g_v, rid_v, rows_v, gath_v, vrows_v, sem):
    cid = lax.axis_index("c")
    sid = lax.axis_index("s")
    wid = sid * NC + cid
    c = wid // GENE_BLOCKS
    gb = wid % GENE_BLOCKS

    pltpu.sync_copy(spots_hbm, spots_v)
    pltpu.sync_copy(genes_hbm.at[pl.ds(gb * ROWS_PER_W, ROWS_PER_W)], g_v)
    for q in range(ROWS_PER_W // L):
        rid_v[pl.ds(q * L, L)] = g_v[pl.ds(q * L, L)] + c * G

    pltpu.async_copy(v_hbm.at[rid_v], vrows_v, sem).wait()
    pltpu.sync_copy(vrows_v, v_out.at[pl.ds(wid * ROWS_PER_W, ROWS_PER_W)])

    pltpu.sync_copy(w_hbm.at[pl.ds(wid * W_ROWS_PER_W, W_ROWS_PER_W)],
                    rows_v.at[pl.ds(0, W_ROWS_PER_W)])
    for r in range(W_ROWS_PER_W):
        _gather_row(spots_v, rows_v, gath_v, r)
    pltpu.sync_copy(gath_v.at[pl.ds(0, W_ROWS_PER_W)],
                    w_out.at[pl.ds(wid * W_ROWS_PER_W, W_ROWS_PER_W)])

    def batch(t, _):
        pltpu.async_copy(lr_hbm.at[rid_v.at[pl.ds(t * RB, RB)]], rows_v, sem).wait()
        for r in range(RB):
            _gather_row(spots_v, rows_v, gath_v, r)
        pltpu.sync_copy(gath_v, lr_out.at[pl.ds(wid * ROWS_PER_W + t * RB, RB)])
        return 0
    lax.fori_loop(0, NBATCH, batch, 0)


_sc_gather = functools.partial(
    pl.kernel,
    mesh=plsc.VectorSubcoreMesh(core_axis_name="c", subcore_axis_name="s"),
    compiler_params=pltpu.CompilerParams(
        needs_layout_passes=False, use_tc_tiling_on_sc=False
    ),
    out_type=[
        jax.ShapeDtypeStruct((C * G_SEL, S_SEL), jnp.float32),
        jax.ShapeDtypeStruct((C * P, S_SEL), jnp.float32),
        jax.ShapeDtypeStruct((C * G_SEL, P), jnp.float32),
    ],
    scratch_types=[
        pltpu.VMEM((S_SEL,), jnp.int32),
        pltpu.VMEM((ROWS_PER_W,), jnp.int32),
        pltpu.VMEM((ROWS_PER_W,), jnp.int32),
        pltpu.VMEM((RB, S), jnp.float32),
        pltpu.VMEM((RB, S_SEL), jnp.float32),
        pltpu.VMEM((ROWS_PER_W, P), jnp.float32),
        pltpu.SemaphoreType.DMA,
    ],
)(_sc_body)


BG = 64


def _tc_body(lr_ref, w_ref, v_ref, o_ref):
    acc = jnp.zeros((BG, S_SEL), jnp.float32)
    for c in range(C):
        prog = jnp.dot(v_ref[c], w_ref[c], preferred_element_type=jnp.float32)
        acc = acc + jnp.exp(lr_ref[c] + prog)
    o_ref[...] = acc


_tc_combine = pl.pallas_call(
    _tc_body,
    grid=(G_SEL // BG,),
    in_specs=[
        pl.BlockSpec((C, BG, S_SEL), lambda i: (0, i, 0)),
        pl.BlockSpec((C, P, S_SEL), lambda i: (0, 0, 0)),
        pl.BlockSpec((C, BG, P), lambda i: (0, i, 0)),
    ],
    out_specs=pl.BlockSpec((BG, S_SEL), lambda i: (i, 0)),
    out_shape=jax.ShapeDtypeStruct((G_SEL, S_SEL), jnp.float32),
)


def kernel(log_rates, W, V, spots, genes):
    lr_tab = log_rates.reshape(C * G, S)
    w_tab = W.reshape(C * P, S)
    v_tab = V.reshape(C * G, P)
    spots32 = spots.astype(jnp.int32)
    genes32 = genes.astype(jnp.int32)

    lr_sel, w_sel, v_sel = _sc_gather(lr_tab, w_tab, v_tab, spots32, genes32)
    return _tc_combine(
        lr_sel.reshape(C, G_SEL, S_SEL),
        w_sel.reshape(C, P, S_SEL),
        v_sel.reshape(C, G_SEL, P),
    )
